# Initial kernel scaffold; baseline (speedup 1.0000x reference)
import jax
import jax.numpy as jnp
from jax.experimental import pallas as pl


def kernel(input_ids, position_ids, word_table, pos_table):
    raise NotImplementedError("write your pallas kernel here")


# speedup vs baseline: 1.6867x; 1.6867x over previous
import functools

import jax
import jax.numpy as jnp
from jax.experimental import pallas as pl
from jax.experimental.pallas import tpu as pltpu
from jax.experimental.pallas import tpu_sc as plsc

_B = 4096
_L = 200
_EMB = 64
_N = _B * _L
_W = 128


def _emb_kernel(wt_hbm, pt_hbm, ids_hbm, pids_hbm, out_hbm, pbuf):
    def body(i_vmem, p_vmem, o_vmem):
        pltpu.sync_copy(wt_hbm.at[i_vmem.at[0]], o_vmem)
        pltpu.sync_copy(pt_hbm.at[p_vmem.at[0]], pbuf)

        @pl.loop(0, _W)
        def _(r):
            for c in range(_EMB // 16):
                sl = pl.ds(c * 16, 16)
                o_vmem[r, sl] += pbuf[r, sl]

    pltpu.emit_pipeline(
        body,
        grid=(_N // _W,),
        in_specs=[
            pl.BlockSpec((1, _W), lambda i: (0, i)),
            pl.BlockSpec((1, _W), lambda i: (0, i)),
        ],
        out_specs=[pl.BlockSpec((_W, _EMB), lambda i: (i, 0))],
        core_axis_name=("c", "s"),
        dimension_semantics=(pltpu.PARALLEL,),
    )(ids_hbm, pids_hbm, out_hbm)


def kernel(input_ids, position_ids, word_table, pos_table):
    ids = input_ids.astype(jnp.int32).reshape(1, _N)
    pids = position_ids.astype(jnp.int32).reshape(1, _N)
    mesh = plsc.VectorSubcoreMesh(core_axis_name="c", subcore_axis_name="s")
    run = pl.kernel(
        _emb_kernel,
        out_type=jax.ShapeDtypeStruct((_N, _EMB), jnp.float32),
        mesh=mesh,
        scratch_types=[pltpu.VMEM((_W, _EMB), jnp.float32)],
        compiler_params=pltpu.CompilerParams(use_tc_tiling_on_sc=False),
    )
    out = run(word_table, pos_table, ids, pids)
    return out.reshape(_B, _L, _EMB)


# speedup vs baseline: 1.9527x; 1.1577x over previous
import jax
import jax.numpy as jnp
from jax import lax
from jax.experimental import pallas as pl
from jax.experimental.pallas import tpu as pltpu
from jax.experimental.pallas import tpu_sc as plsc

_B = 4096
_L = 200
_EMB = 64
_N = _B * _L
_NW = 32
_R = _N // _NW
_C = 128
_NCH = _R // _C
_NBUF = 4


def _emb_kernel(wt, pt, ids, pids, out, iv, piv,
                w0, w1, w2, w3, p0, p1, p2, p3,
                gw0, gw1, gw2, gw3, gp0, gp1, gp2, gp3,
                os0, os1, os2, os3):
    wbufs = [w0, w1, w2, w3]
    pbufs = [p0, p1, p2, p3]
    gw = [gw0, gw1, gw2, gw3]
    gp = [gp0, gp1, gp2, gp3]
    osm = [os0, os1, os2, os3]

    wid = lax.axis_index("s") * 2 + lax.axis_index("c")
    base = wid * _R

    pltpu.sync_copy(ids.at[pl.ds(base, _R)], iv)
    pltpu.sync_copy(pids.at[pl.ds(base, _R)], piv)

    @pl.loop(0, _NCH)
    def _(k):
        sl = pl.ds(k * _C, _C)
        pltpu.sync_copy(wt.at[iv.at[sl]], wbufs[0])
        pltpu.sync_copy(pt.at[piv.at[sl]], pbufs[0])

        @pl.loop(0, _C)
        def _(r):
            for c in range(_EMB // 16):
                csl = pl.ds(c * 16, 16)
                wbufs[0][r, csl] += pbufs[0][r, csl]

        pltpu.sync_copy(wbufs[0], out.at[pl.ds(base + k * _C, _C)])


def kernel(input_ids, position_ids, word_table, pos_table):
    ids = input_ids.astype(jnp.int32).reshape(_N)
    pids = position_ids.astype(jnp.int32).reshape(_N)
    mesh = plsc.VectorSubcoreMesh(core_axis_name="c", subcore_axis_name="s")
    run = pl.kernel(
        _emb_kernel,
        out_type=jax.ShapeDtypeStruct((_N, _EMB), jnp.float32),
        mesh=mesh,
        scratch_types=(
            [pltpu.VMEM((_R,), jnp.int32)] * 2
            + [pltpu.VMEM((_C, _EMB), jnp.float32)] * (2 * _NBUF)
            + [pltpu.SemaphoreType.DMA] * (3 * _NBUF)
        ),
        compiler_params=pltpu.CompilerParams(use_tc_tiling_on_sc=False),
    )
    out = run(word_table, pos_table, ids, pids)
    return out.reshape(_B, _L, _EMB)


# speedup vs baseline: 1.9960x; 1.0221x over previous
import jax
import jax.numpy as jnp
from jax import lax
from jax.experimental import pallas as pl
from jax.experimental.pallas import tpu as pltpu
from jax.experimental.pallas import tpu_sc as plsc

_B = 4096
_L = 200
_EMB = 64
_N = _B * _L
_NW = 32
_R = _N // _NW
_C = 128
_NCH = _R // _C
_NBUF = 4


def _emb_kernel(wt, pt, ids, pids, out, iv, piv,
                w0, w1, w2, w3, p0, p1, p2, p3,
                gw0, gw1, gw2, gw3, gp0, gp1, gp2, gp3,
                os0, os1, os2, os3):
    wbufs = [w0, w1, w2, w3]
    pbufs = [p0, p1, p2, p3]
    gw = [gw0, gw1, gw2, gw3]
    gp = [gp0, gp1, gp2, gp3]
    osm = [os0, os1, os2, os3]

    wid = lax.axis_index("s") * 2 + lax.axis_index("c")
    base = wid * _R

    pltpu.sync_copy(ids.at[pl.ds(base, _R)], iv)
    pltpu.sync_copy(pids.at[pl.ds(base, _R)], piv)

    def gather_desc(k, b):
        sl = pl.ds(k * _C, _C)
        return (pltpu.make_async_copy(wt.at[iv.at[sl]], wbufs[b], gw[b]),
                pltpu.make_async_copy(pt.at[piv.at[sl]], pbufs[b], gp[b]))

    def out_desc(k, b):
        return pltpu.make_async_copy(
            wbufs[b], out.at[pl.ds(base + k * _C, _C)], osm[b])

    for b in range(2):
        dw, dp = gather_desc(b, b)
        dw.start()
        dp.start()

    @pl.loop(0, _NCH // _NBUF)
    def _(ko):
        for b in range(_NBUF):
            k = ko * _NBUF + b
            bn = (b + 2) % _NBUF

            @pl.when(k + 2 < _NCH)
            def _():
                @pl.when(k >= 2)
                def _():
                    out_desc(k - 2, bn).wait()
                dw, dp = gather_desc(k + 2, bn)
                dw.start()
                dp.start()

            dw, dp = gather_desc(k, b)
            dw.wait()
            dp.wait()

            @pl.loop(0, _C)
            def _(r):
                for c in range(_EMB // 16):
                    sl = pl.ds(c * 16, 16)
                    wbufs[b][r, sl] += pbufs[b][r, sl]

            out_desc(k, b).start()

    for b in range(_NBUF):
        out_desc(_NCH - _NBUF + b, b).wait()


def kernel(input_ids, position_ids, word_table, pos_table):
    ids = input_ids.astype(jnp.int32).reshape(_N)
    pids = position_ids.astype(jnp.int32).reshape(_N)
    mesh = plsc.VectorSubcoreMesh(core_axis_name="c", subcore_axis_name="s")
    run = pl.kernel(
        _emb_kernel,
        out_type=jax.ShapeDtypeStruct((_N, _EMB), jnp.float32),
        mesh=mesh,
        scratch_types=(
            [pltpu.VMEM((_R,), jnp.int32)] * 2
            + [pltpu.VMEM((_C, _EMB), jnp.float32)] * (2 * _NBUF)
            + [pltpu.SemaphoreType.DMA] * (3 * _NBUF)
        ),
        compiler_params=pltpu.CompilerParams(use_tc_tiling_on_sc=False),
    )
    out = run(word_table, pos_table, ids, pids)
    return out.reshape(_B, _L, _EMB)


# speedup vs baseline: 2.8572x; 1.4315x over previous
import jax
import jax.numpy as jnp
from jax import lax
from jax.experimental import pallas as pl
from jax.experimental.pallas import tpu as pltpu
from jax.experimental.pallas import tpu_sc as plsc

_B = 4096
_L = 200
_EMB = 64
_N = _B * _L
_NW = 32
_R = _N // _NW
_C = 128
_NCH = _R // _C
_NBUF = 4


def _emb_kernel(wt, pt, ids, pids, out, iv, piv, ptab,
                w0, w1, w2, w3, p0, p1, p2, p3,
                gw0, gw1, gw2, gw3, gp0, gp1, gp2, gp3,
                os0, os1, os2, os3):
    wbufs = [w0, w1, w2, w3]
    pbufs = [p0, p1, p2, p3]
    gw = [gw0, gw1, gw2, gw3]
    gp = [gp0, gp1, gp2, gp3]
    osm = [os0, os1, os2, os3]

    wid = lax.axis_index("s") * 2 + lax.axis_index("c")
    base = wid * _R

    pltpu.sync_copy(ids.at[pl.ds(base, _R)], iv)
    pltpu.sync_copy(pids.at[pl.ds(base, _R)], piv)
    @pl.when(lax.axis_index("s") == 0)
    def _():
        pltpu.sync_copy(pt, ptab)
    plsc.subcore_barrier()

    def gather_desc(k, b):
        sl = pl.ds(k * _C, _C)
        return (pltpu.make_async_copy(wt.at[iv.at[sl]], wbufs[b], gw[b]),
                pltpu.make_async_copy(ptab.at[piv.at[sl]], pbufs[b], gp[b]))

    def out_desc(k, b):
        return pltpu.make_async_copy(
            wbufs[b], out.at[pl.ds(base + k * _C, _C)], osm[b])

    for b in range(2):
        dw, dp = gather_desc(b, b)
        dw.start()
        dp.start()

    @pl.loop(0, _NCH // _NBUF)
    def _(ko):
        for b in range(_NBUF):
            k = ko * _NBUF + b
            bn = (b + 2) % _NBUF

            @pl.when(k + 2 < _NCH)
            def _():
                @pl.when(k >= 2)
                def _():
                    out_desc(k - 2, bn).wait()
                dw, dp = gather_desc(k + 2, bn)
                dw.start()
                dp.start()

            dw, dp = gather_desc(k, b)
            dw.wait()
            dp.wait()

            @pl.loop(0, _C)
            def _(r):
                for c in range(_EMB // 16):
                    sl = pl.ds(c * 16, 16)
                    wbufs[b][r, sl] += pbufs[b][r, sl]

            out_desc(k, b).start()

    for b in range(_NBUF):
        out_desc(_NCH - _NBUF + b, b).wait()


def kernel(input_ids, position_ids, word_table, pos_table):
    ids = input_ids.astype(jnp.int32).reshape(_N)
    pids = position_ids.astype(jnp.int32).reshape(_N)
    mesh = plsc.VectorSubcoreMesh(core_axis_name="c", subcore_axis_name="s")
    run = pl.kernel(
        _emb_kernel,
        out_type=jax.ShapeDtypeStruct((_N, _EMB), jnp.float32),
        mesh=mesh,
        scratch_types=(
            [pltpu.VMEM((_R,), jnp.int32)] * 2
            + [pltpu.VMEM_SHARED((_L, _EMB), jnp.float32)]
            + [pltpu.VMEM((_C, _EMB), jnp.float32)] * (2 * _NBUF)
            + [pltpu.SemaphoreType.DMA] * (3 * _NBUF)
        ),
        compiler_params=pltpu.CompilerParams(use_tc_tiling_on_sc=False),
    )
    out = run(word_table, pos_table, ids, pids)
    return out.reshape(_B, _L, _EMB)


# speedup vs baseline: 2.8616x; 1.0015x over previous
import jax
import jax.numpy as jnp
from jax import lax
from jax.experimental import pallas as pl
from jax.experimental.pallas import tpu as pltpu
from jax.experimental.pallas import tpu_sc as plsc

_B = 4096
_L = 200
_EMB = 64
_N = _B * _L
_NW = 32
_R = _N // _NW
_C = 200
_NCH = _R // _C
_NBUF = 4


def _emb_kernel(wt, pt, ids, pids, out, ptab,
                w0, w1, w2, w3, p0, p1, p2, p3,
                i0, i1, i2, i3, q0, q1, q2, q3,
                gw0, gw1, gw2, gw3, gp0, gp1, gp2, gp3,
                os0, os1, os2, os3, is0, is1, is2, is3):
    wbufs = [w0, w1, w2, w3]
    pbufs = [p0, p1, p2, p3]
    islt = [i0, i1, i2, i3]
    qslt = [q0, q1, q2, q3]
    gw = [gw0, gw1, gw2, gw3]
    gp = [gp0, gp1, gp2, gp3]
    osm = [os0, os1, os2, os3]
    ism = [is0, is1, is2, is3]

    wid = lax.axis_index("s") * 2 + lax.axis_index("c")
    base = wid * _R

    @pl.when(lax.axis_index("s") == 0)
    def _():
        pltpu.sync_copy(pt, ptab)
    plsc.subcore_barrier()

    def idx_descs(k, j):
        sl = pl.ds(base + k * _C, _C)
        return (pltpu.make_async_copy(ids.at[sl], islt[j], ism[j]),
                pltpu.make_async_copy(pids.at[sl], qslt[j], ism[j]))

    def gather_descs(b):
        return (pltpu.make_async_copy(wt.at[islt[b]], wbufs[b], gw[b]),
                pltpu.make_async_copy(ptab.at[qslt[b]], pbufs[b], gp[b]))

    def out_desc(k, b):
        return pltpu.make_async_copy(
            wbufs[b], out.at[pl.ds(base + k * _C, _C)], osm[b])

    for k in range(4):
        da, db = idx_descs(k, k)
        da.start()
        db.start()
    for b in range(2):
        da, db = idx_descs(b, b)
        da.wait()
        db.wait()
        dw, dp = gather_descs(b)
        dw.start()
        dp.start()

    @pl.loop(0, _NCH // _NBUF)
    def _(ko):
        for b in range(_NBUF):
            k = ko * _NBUF + b
            bn = (b + 2) % _NBUF

            @pl.when(k + 2 < _NCH)
            def _():
                da, db = idx_descs(k + 2, bn)
                da.wait()
                db.wait()

                @pl.when(k >= 2)
                def _():
                    out_desc(k - 2, bn).wait()
                dw, dp = gather_descs(bn)
                dw.start()
                dp.start()

            dw, dp = gather_descs(b)
            dw.wait()
            dp.wait()

            @pl.when(k + 4 < _NCH)
            def _():
                da, db = idx_descs(k + 4, b)
                da.start()
                db.start()

            @pl.loop(0, _C)
            def _(r):
                for c in range(_EMB // 16):
                    sl = pl.ds(c * 16, 16)
                    wbufs[b][r, sl] += pbufs[b][r, sl]

            out_desc(k, b).start()

    for b in range(_NBUF):
        out_desc(_NCH - _NBUF + b, b).wait()


def kernel(input_ids, position_ids, word_table, pos_table):
    ids = input_ids.astype(jnp.int32).reshape(_N)
    pids = position_ids.astype(jnp.int32).reshape(_N)
    mesh = plsc.VectorSubcoreMesh(core_axis_name="c", subcore_axis_name="s")
    run = pl.kernel(
        _emb_kernel,
        out_type=jax.ShapeDtypeStruct((_N, _EMB), jnp.float32),
        mesh=mesh,
        scratch_types=(
            [pltpu.VMEM_SHARED((_L, _EMB), jnp.float32)]
            + [pltpu.VMEM((_C, _EMB), jnp.float32)] * (2 * _NBUF)
            + [pltpu.VMEM((_C,), jnp.int32)] * (2 * _NBUF)
            + [pltpu.SemaphoreType.DMA] * (4 * _NBUF)
        ),
        compiler_params=pltpu.CompilerParams(use_tc_tiling_on_sc=False),
    )
    out = run(word_table, pos_table, ids, pids)
    return out.reshape(_B, _L, _EMB)


# speedup vs baseline: 2.8696x; 1.0028x over previous
import jax
import jax.numpy as jnp
from jax import lax
from jax.experimental import pallas as pl
from jax.experimental.pallas import tpu as pltpu
from jax.experimental.pallas import tpu_sc as plsc

_B = 4096
_L = 200
_EMB = 64
_N = _B * _L
_NW = 32
_R = _N // _NW
_C = 200
_NCH = _R // _C
_NBUF = 4


def _emb_kernel(wt, pt, ids, pids, out, ptab,
                w0, w1, w2, w3, p0, p1, p2, p3,
                i0, i1, i2, i3, q0, q1, q2, q3,
                gw0, gw1, gw2, gw3, gp0, gp1, gp2, gp3,
                os0, os1, os2, os3, is0, is1, is2, is3):
    wbufs = [w0, w1, w2, w3]
    pbufs = [p0, p1, p2, p3]
    islt = [i0, i1, i2, i3]
    qslt = [q0, q1, q2, q3]
    gw = [gw0, gw1, gw2, gw3]
    gp = [gp0, gp1, gp2, gp3]
    osm = [os0, os1, os2, os3]
    ism = [is0, is1, is2, is3]

    wid = lax.axis_index("s") * 2 + lax.axis_index("c")
    base = wid * _R

    @pl.when(lax.axis_index("s") == 0)
    def _():
        pltpu.sync_copy(pt, ptab)
    plsc.subcore_barrier()

    def idx_descs(k, j):
        sl = pl.ds(base + k * _C, _C)
        return (pltpu.make_async_copy(ids.at[sl], islt[j], ism[j]),
                pltpu.make_async_copy(pids.at[sl], qslt[j], ism[j]))

    _S = 5
    _CS = _C // _S

    def gather_descs(b):
        subs = [pltpu.make_async_copy(
                    wt.at[islt[b].at[pl.ds(s * _CS, _CS)]],
                    wbufs[b].at[pl.ds(s * _CS, _CS)], gw[b])
                for s in range(_S)]
        subs.append(pltpu.make_async_copy(ptab.at[qslt[b]], pbufs[b], gp[b]))
        return subs

    def out_desc(k, b):
        return pltpu.make_async_copy(
            wbufs[b], out.at[pl.ds(base + k * _C, _C)], osm[b])

    for k in range(4):
        da, db = idx_descs(k, k)
        da.start()
        db.start()
    for b in range(2):
        da, db = idx_descs(b, b)
        da.wait()
        db.wait()
        for d in gather_descs(b):
            d.start()

    @pl.loop(0, _NCH // _NBUF)
    def _(ko):
        for b in range(_NBUF):
            k = ko * _NBUF + b
            bn = (b + 2) % _NBUF

            @pl.when(k + 2 < _NCH)
            def _():
                da, db = idx_descs(k + 2, bn)
                da.wait()
                db.wait()

                @pl.when(k >= 2)
                def _():
                    out_desc(k - 2, bn).wait()
                for d in gather_descs(bn):
                    d.start()

            for d in gather_descs(b):
                d.wait()

            @pl.when(k + 4 < _NCH)
            def _():
                da, db = idx_descs(k + 4, b)
                da.start()
                db.start()

            @pl.loop(0, _C)
            def _(r):
                for c in range(_EMB // 16):
                    sl = pl.ds(c * 16, 16)
                    wbufs[b][r, sl] += pbufs[b][r, sl]

            out_desc(k, b).start()

    for b in range(_NBUF):
        out_desc(_NCH - _NBUF + b, b).wait()


def kernel(input_ids, position_ids, word_table, pos_table):
    ids = input_ids.astype(jnp.int32).reshape(_N)
    pids = position_ids.astype(jnp.int32).reshape(_N)
    mesh = plsc.VectorSubcoreMesh(core_axis_name="c", subcore_axis_name="s")
    run = pl.kernel(
        _emb_kernel,
        out_type=jax.ShapeDtypeStruct((_N, _EMB), jnp.float32),
        mesh=mesh,
        scratch_types=(
            [pltpu.VMEM_SHARED((_L, _EMB), jnp.float32)]
            + [pltpu.VMEM((_C, _EMB), jnp.float32)] * (2 * _NBUF)
            + [pltpu.VMEM((_C,), jnp.int32)] * (2 * _NBUF)
            + [pltpu.SemaphoreType.DMA] * (4 * _NBUF)
        ),
        compiler_params=pltpu.CompilerParams(use_tc_tiling_on_sc=False),
    )
    out = run(word_table, pos_table, ids, pids)
    return out.reshape(_B, _L, _EMB)
